# Initial kernel scaffold; baseline (speedup 1.0000x reference)
import jax
import jax.numpy as jnp
from jax.experimental import pallas as pl


def kernel(x, embedding_table, positional_embedding):
    raise NotImplementedError("write your pallas kernel here")


# speedup vs baseline: 2.8492x; 2.8492x over previous
import jax
import jax.numpy as jnp
from jax import lax
from jax.experimental import pallas as pl
from jax.experimental.pallas import tpu as pltpu
from jax.experimental.pallas import tpu_sc as plsc

N_VOCAB = 100000
N_EMBD = 128
N_TOKENS = 200
BATCH = 1024

_NC = 2
_NS = 16
_NW = _NC * _NS
_BPW = BATCH // _NW
_H = N_TOKENS // 2


def _body(x_ref, tab_ref, p0_ref, p1_ref, out_ref,
          idx_v, p0_sh, p1_sh, buf0_v, buf1_v, sem0, sem1):
    sid = lax.axis_index("s")
    wid = sid * _NC + lax.axis_index("c")
    pltpu.sync_copy(x_ref.at[pl.ds(wid * _BPW, _BPW)], idx_v)

    @pl.when(sid == 0)
    def _():
        pltpu.sync_copy(p0_ref, p0_sh)
        pltpu.sync_copy(p1_ref, p1_sh)

    plsc.subcore_barrier()

    def step(j, carry):
        pltpu.sync_copy(p0_sh, buf0_v)
        c0 = pltpu.async_copy(tab_ref.at[idx_v.at[j, 0]], buf0_v, sem0, add=True)
        pltpu.sync_copy(p1_sh, buf1_v)
        c1 = pltpu.async_copy(tab_ref.at[idx_v.at[j, 1]], buf1_v, sem1, add=True)
        c0.wait()
        pltpu.sync_copy(buf0_v, out_ref.at[wid * _BPW + j, 0])
        c1.wait()
        pltpu.sync_copy(buf1_v, out_ref.at[wid * _BPW + j, 1])
        return carry

    lax.fori_loop(0, _BPW, step, 0)


@jax.jit
def kernel(x, embedding_table, positional_embedding):
    x3 = x.reshape(BATCH, 2, _H).astype(jnp.int32)
    p0 = positional_embedding[:_H]
    p1 = positional_embedding[_H:]
    mesh = plsc.VectorSubcoreMesh(
        core_axis_name="c", subcore_axis_name="s",
        num_cores=_NC, num_subcores=_NS)
    out = pl.kernel(
        _body,
        out_type=jax.ShapeDtypeStruct((BATCH, 2, _H, N_EMBD), jnp.float32),
        mesh=mesh,
        scratch_types=[
            pltpu.VMEM((_BPW, 2, _H), jnp.int32),
            pltpu.VMEM_SHARED((_H, N_EMBD), jnp.float32),
            pltpu.VMEM_SHARED((_H, N_EMBD), jnp.float32),
            pltpu.VMEM((_H, N_EMBD), jnp.float32),
            pltpu.VMEM((_H, N_EMBD), jnp.float32),
            pltpu.SemaphoreType.DMA,
            pltpu.SemaphoreType.DMA,
        ],
    )(x3, embedding_table, p0, p1)
    return out.reshape(BATCH, N_TOKENS, N_EMBD)


# speedup vs baseline: 3.1671x; 1.1116x over previous
import jax
import jax.numpy as jnp
from jax import lax
from jax.experimental import pallas as pl
from jax.experimental.pallas import tpu as pltpu
from jax.experimental.pallas import tpu_sc as plsc

N_VOCAB = 100000
N_EMBD = 128
N_TOKENS = 200
BATCH = 1024

_NC = 2
_NS = 16
_NW = _NC * _NS
_BPW = BATCH // _NW
_H = N_TOKENS // 2
_NH = 2 * _BPW
_NBUF = 4


def _body(x_ref, tab_ref, p0_ref, p1_ref, out_ref,
          idx_v, p0_sh, p1_sh,
          b0, b1, b2, b3, sg0, sg1, sg2, sg3, so0, so1, so2, so3):
    sid = lax.axis_index("s")
    wid = sid * _NC + lax.axis_index("c")
    base = wid * _BPW
    bufs = [b0, b1, b2, b3]
    sgs = [sg0, sg1, sg2, sg3]
    sos = [so0, so1, so2, so3]
    pos_sh = [p0_sh, p1_sh]

    pltpu.sync_copy(x_ref.at[pl.ds(wid * _BPW, _BPW)], idx_v)

    @pl.when(sid == 0)
    def _():
        pltpu.sync_copy(p0_ref, p0_sh)
        pltpu.sync_copy(p1_ref, p1_sh)

    plsc.subcore_barrier()

    def launch(j, jj, q):
        pltpu.sync_copy(pos_sh[q % 2], bufs[q])
        pltpu.async_copy(tab_ref.at[idx_v.at[jj, q % 2]], bufs[q],
                         sgs[q], add=True)

    def drain(j, jj, q):
        pltpu.make_async_copy(tab_ref.at[idx_v.at[jj, q % 2]], bufs[q],
                              sgs[q]).wait()
        pltpu.async_copy(bufs[q], out_ref.at[base + jj, q % 2], sos[q])

    def wait_out(j, jj, q):
        pltpu.make_async_copy(bufs[q], out_ref.at[base + jj, q % 2],
                              sos[q]).wait()

    launch(0, 0, 0)
    launch(1, 0, 1)
    launch(2, 1, 2)
    drain(0, 0, 0)
    launch(3, 1, 3)
    drain(1, 0, 1)

    def group(g, carry):
        j0 = 4 * g
        for q in range(4):
            j = j0 + q
            jj = j // 2
            wait_out(j - 4, jj - 2, q)
            launch(j, jj, q)
            drain(j - 2, (j - 2) // 2, (q + 2) % 4)
        return carry

    lax.fori_loop(1, _NH // 4, group, 0)

    drain(_NH - 2, _BPW - 1, 2)
    drain(_NH - 1, _BPW - 1, 3)
    wait_out(_NH - 4, _BPW - 2, 0)
    wait_out(_NH - 3, _BPW - 2, 1)
    wait_out(_NH - 2, _BPW - 1, 2)
    wait_out(_NH - 1, _BPW - 1, 3)


@jax.jit
def kernel(x, embedding_table, positional_embedding):
    x3 = x.reshape(BATCH, 2, _H).astype(jnp.int32)
    p0 = positional_embedding[:_H]
    p1 = positional_embedding[_H:]
    mesh = plsc.VectorSubcoreMesh(
        core_axis_name="c", subcore_axis_name="s",
        num_cores=_NC, num_subcores=_NS)
    out = pl.kernel(
        _body,
        out_type=jax.ShapeDtypeStruct((BATCH, 2, _H, N_EMBD), jnp.float32),
        mesh=mesh,
        scratch_types=[
            pltpu.VMEM((_BPW, 2, _H), jnp.int32),
            pltpu.VMEM_SHARED((_H, N_EMBD), jnp.float32),
            pltpu.VMEM_SHARED((_H, N_EMBD), jnp.float32),
        ] + [pltpu.VMEM((_H, N_EMBD), jnp.float32)] * _NBUF
          + [pltpu.SemaphoreType.DMA] * (2 * _NBUF),
    )(x3, embedding_table, p0, p1)
    return out.reshape(BATCH, N_TOKENS, N_EMBD)
